# baseline (speedup 1.0000x reference)
import jax
import jax.numpy as jnp
from jax import lax
from jax.experimental import pallas as pl
from jax.experimental.pallas import tpu as pltpu
from jax.experimental.pallas import tpu_sc as plsc

NK = 1024
D = 64
DP = 128
CW = 16
B = 16
PIX = 1024
N = B * PIX
NW = 32
RPW = N // NW
GSZ = 128
G = RPW // GSZ
RPT = NK // 16

DECAY = 0.99
EPS = 1e-5
CC = 0.25
RATIO = DECAY / (1.0 - DECAY)


BPS = 4


def _argmin_body(x_ref, w_ref, ecs_ref, idx_ref, xt_ref, n_ref, x2acc_ref):
    i = pl.program_id(0)
    Wm = w_ref[...]
    w2 = jnp.sum(Wm * Wm, axis=1, keepdims=True)
    ey = (lax.broadcasted_iota(jnp.int32, (D, DP), 0)
          == lax.broadcasted_iota(jnp.int32, (D, DP), 1)).astype(jnp.float32)
    colid = lax.broadcasted_iota(jnp.int32, (PIX, DP), 1)
    onecols = ((colid >= D) & (colid < D + CW)).astype(jnp.float32)
    rows = lax.broadcasted_iota(jnp.int32, (NK, PIX), 0)
    x2s = 0.0
    W2 = Wm + Wm
    for bb in range(BPS):
        X = x_ref[bb]
        S2 = lax.dot_general(W2, X, (((1,), (0,)), ((), ())),
                             preferred_element_type=jnp.float32)
        x2 = jnp.sum(X * X, axis=0, keepdims=True)
        dist = (x2 + w2) - S2
        dmin = jnp.min(dist, axis=0, keepdims=True)
        idx = jnp.min(jnp.where(dist == dmin, rows, NK), axis=0)
        idx_ref[bb, 0, :] = idx
        xt_ref[bb] = (lax.dot_general(X, ey, (((0,), (0,)), ((), ())),
                                      preferred_element_type=jnp.float32)
                      + onecols)
        x2s = x2s + jnp.sum(x2)

    @pl.when(i == 0)
    def _():
        n = DECAY * jnp.sum(ecs_ref[...]) + (1.0 - DECAY) * N
        n_ref[0:1] = jnp.full((1, DP), n, jnp.float32)
        x2acc_ref[0, 0] = 0.0

    x2acc_ref[0, 0] += x2s

    @pl.when(i == B // BPS - 1)
    def _():
        n_ref[1:2] = jnp.full((1, DP), x2acc_ref[0, 0], jnp.float32)


def _stage_a(x3, W, ecs8):
    return pl.pallas_call(
        _argmin_body,
        grid=(B // BPS,),
        in_specs=[pl.BlockSpec((BPS, D, PIX), lambda i: (i, 0, 0)),
                  pl.BlockSpec((NK, D), lambda i: (0, 0)),
                  pl.BlockSpec((8, DP), lambda i: (0, 0))],
        out_specs=[pl.BlockSpec((BPS, 1, PIX), lambda i: (i, 0, 0)),
                   pl.BlockSpec((BPS, PIX, DP), lambda i: (i, 0, 0)),
                   pl.BlockSpec((2, DP), lambda i: (0, 0))],
        out_shape=[jax.ShapeDtypeStruct((B, 1, PIX), jnp.int32),
                   jax.ShapeDtypeStruct((B, PIX, DP), jnp.float32),
                   jax.ShapeDtypeStruct((2, DP), jnp.float32)],
        scratch_shapes=[pltpu.SMEM((1, 1), jnp.float32)],
    )(x3, W, ecs8)


def _scatter_body(x_hbm, idx_hbm, zdw_hbm, emaw_hbm, ecsb_hbm, dwp_hbm,
                  idx_v, rows_v, init_v, ecsb_v, dw_sh, sem, rsem):
    c = lax.axis_index("c")
    s = lax.axis_index("s")
    w = s * 2 + c
    base = s * RPT
    rcp = pltpu.async_copy(x_hbm.at[w // 2, pl.ds((w % 2) * RPW, RPW)],
                           rows_v, rsem)
    pltpu.sync_copy(idx_hbm.at[w], idx_v)

    @pl.when(c == 0)
    def _():
        pltpu.sync_copy(emaw_hbm.at[pl.ds(base, RPT)], init_v)
        pltpu.sync_copy(ecsb_hbm.at[pl.ds(base, RPT)], ecsb_v)

        def _init_row(r, carry):
            for k in range(D // 16):
                sl = pl.ds(k * 16, 16)
                init_v[r, sl] = init_v[r, sl] * RATIO
            init_v[r, pl.ds(D, CW)] = ecsb_v[r, pl.ds(0, CW)] * RATIO
            return carry

        lax.fori_loop(0, RPT, _init_row, 0)
        pltpu.sync_copy(init_v, dw_sh.at[pl.ds(base, RPT)])

    @pl.when(c != 0)
    def _():
        pltpu.sync_copy(zdw_hbm.at[pl.ds(base, RPT)],
                        dw_sh.at[pl.ds(base, RPT)])

    plsc.subcore_barrier()
    rcp.wait()
    cps = [pltpu.async_copy(rows_v.at[pl.ds(g * GSZ, GSZ)],
                            dw_sh.at[idx_v.at[g]], sem, add=True)
           for g in range(G)]
    for cp in cps:
        cp.wait()
    plsc.subcore_barrier()
    pltpu.sync_copy(dw_sh.at[pl.ds(base, RPT)],
                    dwp_hbm.at[c, pl.ds(base, RPT)])


def _stage_b(xt, idx_g, zdw, emaw_p, ecs_bc):
    mesh = plsc.VectorSubcoreMesh(core_axis_name="c", subcore_axis_name="s")
    f = pl.kernel(
        _scatter_body,
        out_type=jax.ShapeDtypeStruct((2, NK, DP), jnp.float32),
        mesh=mesh,
        scratch_types=[pltpu.VMEM((G, GSZ), jnp.int32),
                       pltpu.VMEM((RPW, DP), jnp.float32),
                       pltpu.VMEM((RPT, DP), jnp.float32),
                       pltpu.VMEM((RPT, CW), jnp.float32),
                       pltpu.VMEM_SHARED((NK, DP), jnp.float32),
                       pltpu.SemaphoreType.DMA,
                       pltpu.SemaphoreType.DMA],
    )
    return f(xt, idx_g, zdw, emaw_p, ecs_bc)


def _ema_gather_body(dwp_hbm, n_hbm, idx_hbm, emaw_hbm, q_hbm, parts_hbm,
                     idx_v, rows_v, dw0_v, dw1_v, emaw_v, n_row, wnew_v,
                     parts_v, wnew_sh, sem, wsem):
    c = lax.axis_index("c")
    s = lax.axis_index("s")
    w = s * 2 + c
    base = s * RPT
    pltpu.sync_copy(idx_hbm.at[w], idx_v)
    pltpu.sync_copy(dwp_hbm.at[0, pl.ds(base, RPT)], dw0_v)
    pltpu.sync_copy(dwp_hbm.at[1, pl.ds(base, RPT)], dw1_v)
    pltpu.sync_copy(emaw_hbm.at[pl.ds(base, RPT)], emaw_v)
    pltpu.sync_copy(n_hbm.at[pl.ds(0, 1)], n_row)
    nv = n_row[0, pl.ds(0, 16)]
    den = nv + NK * EPS
    OM = 1.0 - DECAY

    def _row_step(r, carry):
        vqx, vq2 = carry
        cnt = dw0_v[r, pl.ds(D, 16)] + dw1_v[r, pl.ds(D, 16)]
        ncs = OM * cnt
        cs = (ncs + EPS) / den * nv
        iv = OM / cs
        w2r = jnp.zeros((16,), jnp.float32)
        for k in range(D // 16):
            sl = pl.ds(k * 16, 16)
            acc = dw0_v[r, sl] + dw1_v[r, sl]
            wn = acc * iv
            wnew_v[r, sl] = wn
            vqx = vqx + wn * (acc - RATIO * emaw_v[r, sl])
            w2r = w2r + wn * wn
        vq2 = vq2 + cnt * w2r
        return (vqx, vq2)

    vqx, vq2 = lax.fori_loop(0, RPT, _row_step,
                             (jnp.zeros((16,), jnp.float32),
                              jnp.zeros((16,), jnp.float32)))
    parts_v[0, pl.ds(0, 16)] = vqx
    parts_v[1, pl.ds(0, 16)] = vq2
    pltpu.sync_copy(parts_v, parts_hbm.at[w])
    pltpu.sync_copy(wnew_v, wnew_sh.at[pl.ds(base, RPT)])
    plsc.subcore_barrier()

    cps = [pltpu.async_copy(wnew_sh.at[idx_v.at[g]],
                            rows_v.at[pl.ds(g * GSZ, GSZ)], sem)
           for g in range(G)]
    wcps = []
    for g in range(G):
        cps[g].wait()
        wcps.append(pltpu.async_copy(
            rows_v.at[pl.ds(g * GSZ, GSZ)],
            q_hbm.at[w // 2, pl.ds((w % 2) * RPW + g * GSZ, GSZ)], wsem))
    for cp in wcps:
        cp.wait()


def _stage_c(dwp, nsum, idx_g, emaw_p):
    mesh = plsc.VectorSubcoreMesh(core_axis_name="c", subcore_axis_name="s")
    f = pl.kernel(
        _ema_gather_body,
        out_type=[jax.ShapeDtypeStruct((B, PIX, DP), jnp.float32),
                  jax.ShapeDtypeStruct((NW, 2, 16), jnp.float32)],
        mesh=mesh,
        scratch_types=[pltpu.VMEM((G, GSZ), jnp.int32),
                       pltpu.VMEM((RPW, DP), jnp.float32),
                       pltpu.VMEM((RPT, DP), jnp.float32),
                       pltpu.VMEM((RPT, DP), jnp.float32),
                       pltpu.VMEM((RPT, DP), jnp.float32),
                       pltpu.VMEM((1, DP), jnp.float32),
                       pltpu.VMEM((RPT, DP), jnp.float32),
                       pltpu.VMEM((2, 16), jnp.float32),
                       pltpu.VMEM_SHARED((NK, DP), jnp.float32),
                       pltpu.SemaphoreType.DMA,
                       pltpu.SemaphoreType.DMA],
    )
    return f(dwp, nsum, idx_g, emaw_p)


def _finish_body(n_ref, parts_ref, loss_ref):
    x2tot = n_ref[1, 0]
    qx = 0.5 * jnp.sum(parts_ref[:, 0, :])
    q2 = 0.5 * jnp.sum(parts_ref[:, 1, :])
    loss_ref[0, 0] = (q2 - 2.0 * qx + x2tot) * (CC / (N * D))


def _stage_f(nsum, parts):
    return pl.pallas_call(
        _finish_body,
        in_specs=[pl.BlockSpec(memory_space=pltpu.SMEM),
                  pl.BlockSpec((NW, 2, 16), lambda: (0, 0, 0))],
        out_specs=pl.BlockSpec(memory_space=pltpu.SMEM),
        out_shape=jax.ShapeDtypeStruct((1, 1), jnp.float32),
    )(nsum, parts)


def kernel(inputs, W, ema_cluster_size, ema_w):
    x3 = inputs.reshape(B, D, PIX)
    idx3, xt, nsum = _stage_a(x3, W, ema_cluster_size.reshape(8, DP))
    idx_g = idx3.reshape(NW, G, GSZ)
    zdw = jnp.zeros((NK, DP), jnp.float32)
    emaw_p = jnp.pad(ema_w, ((0, 0), (0, DP - D)))
    ecs_bc = jnp.broadcast_to(ema_cluster_size[:, None], (NK, CW))
    dwp = _stage_b(xt, idx_g, zdw, emaw_p, ecs_bc)
    q3, parts = _stage_c(dwp, nsum, idx_g, emaw_p)
    loss2 = _stage_f(nsum, parts)
    qout = q3[:, :, :D].transpose(0, 2, 1).reshape(B, D, 32, 32)
    return (qout, loss2.reshape(()), idx3.reshape(B, 32, 32))


# speedup vs baseline: 1.0063x; 1.0063x over previous
import jax
import jax.numpy as jnp
from jax import lax
from jax.experimental import pallas as pl
from jax.experimental.pallas import tpu as pltpu
from jax.experimental.pallas import tpu_sc as plsc

NK = 1024
D = 64
DP = 128
CW = 16
B = 16
PIX = 1024
N = B * PIX
NW = 32
RPW = N // NW
GSZ = 128
G = RPW // GSZ
RPT = NK // 16

DECAY = 0.99
EPS = 1e-5
CC = 0.25
RATIO = DECAY / (1.0 - DECAY)


BPS = 2


def _argmin_body(x_ref, w_ref, ecs_ref, idx_ref, xt_ref, n_ref, x2acc_ref):
    i = pl.program_id(0)
    Wm = w_ref[...]
    w2 = jnp.sum(Wm * Wm, axis=1, keepdims=True)
    ey = (lax.broadcasted_iota(jnp.int32, (D, DP), 0)
          == lax.broadcasted_iota(jnp.int32, (D, DP), 1)).astype(jnp.float32)
    colid = lax.broadcasted_iota(jnp.int32, (PIX, DP), 1)
    onecols = ((colid >= D) & (colid < D + CW)).astype(jnp.float32)
    rows = lax.broadcasted_iota(jnp.int32, (NK, PIX), 0)
    x2s = 0.0
    W2 = Wm + Wm
    for bb in range(BPS):
        X = x_ref[bb]
        S2 = lax.dot_general(W2, X, (((1,), (0,)), ((), ())),
                             preferred_element_type=jnp.float32)
        x2 = jnp.sum(X * X, axis=0, keepdims=True)
        dist = (x2 + w2) - S2
        dmin = jnp.min(dist, axis=0, keepdims=True)
        idx = jnp.min(jnp.where(dist == dmin, rows, NK), axis=0)
        idx_ref[bb, 0, :] = idx
        xt_ref[bb] = (lax.dot_general(X, ey, (((0,), (0,)), ((), ())),
                                      preferred_element_type=jnp.float32)
                      + onecols)
        x2s = x2s + jnp.sum(x2)

    @pl.when(i == 0)
    def _():
        n = DECAY * jnp.sum(ecs_ref[...]) + (1.0 - DECAY) * N
        n_ref[0:1] = jnp.full((1, DP), n, jnp.float32)
        x2acc_ref[0, 0] = 0.0

    x2acc_ref[0, 0] += x2s

    @pl.when(i == B // BPS - 1)
    def _():
        n_ref[1:2] = jnp.full((1, DP), x2acc_ref[0, 0], jnp.float32)


def _stage_a(x3, W, ecs8):
    return pl.pallas_call(
        _argmin_body,
        grid=(B // BPS,),
        in_specs=[pl.BlockSpec((BPS, D, PIX), lambda i: (i, 0, 0)),
                  pl.BlockSpec((NK, D), lambda i: (0, 0)),
                  pl.BlockSpec((8, DP), lambda i: (0, 0))],
        out_specs=[pl.BlockSpec((BPS, 1, PIX), lambda i: (i, 0, 0)),
                   pl.BlockSpec((BPS, PIX, DP), lambda i: (i, 0, 0)),
                   pl.BlockSpec((2, DP), lambda i: (0, 0))],
        out_shape=[jax.ShapeDtypeStruct((B, 1, PIX), jnp.int32),
                   jax.ShapeDtypeStruct((B, PIX, DP), jnp.float32),
                   jax.ShapeDtypeStruct((2, DP), jnp.float32)],
        scratch_shapes=[pltpu.SMEM((1, 1), jnp.float32)],
    )(x3, W, ecs8)


def _scatter_body(x_hbm, idx_hbm, zdw_hbm, emaw_hbm, ecsb_hbm, dwp_hbm,
                  idx_v, rows_v, init_v, ecsb_v, dw_sh, sem, rsem):
    c = lax.axis_index("c")
    s = lax.axis_index("s")
    w = s * 2 + c
    base = s * RPT
    rcp = pltpu.async_copy(x_hbm.at[w // 2, pl.ds((w % 2) * RPW, RPW)],
                           rows_v, rsem)
    pltpu.sync_copy(idx_hbm.at[w], idx_v)

    @pl.when(c == 0)
    def _():
        pltpu.sync_copy(emaw_hbm.at[pl.ds(base, RPT)], init_v)
        pltpu.sync_copy(ecsb_hbm.at[pl.ds(base, RPT)], ecsb_v)

        def _init_row(r, carry):
            for k in range(D // 16):
                sl = pl.ds(k * 16, 16)
                init_v[r, sl] = init_v[r, sl] * RATIO
            init_v[r, pl.ds(D, CW)] = ecsb_v[r, pl.ds(0, CW)] * RATIO
            return carry

        lax.fori_loop(0, RPT, _init_row, 0)
        pltpu.sync_copy(init_v, dw_sh.at[pl.ds(base, RPT)])

    @pl.when(c != 0)
    def _():
        pltpu.sync_copy(zdw_hbm.at[pl.ds(base, RPT)],
                        dw_sh.at[pl.ds(base, RPT)])

    plsc.subcore_barrier()
    rcp.wait()
    cps = [pltpu.async_copy(rows_v.at[pl.ds(g * GSZ, GSZ)],
                            dw_sh.at[idx_v.at[g]], sem, add=True)
           for g in range(G)]
    for cp in cps:
        cp.wait()
    plsc.subcore_barrier()
    pltpu.sync_copy(dw_sh.at[pl.ds(base, RPT)],
                    dwp_hbm.at[c, pl.ds(base, RPT)])


def _stage_b(xt, idx_g, zdw, emaw_p, ecs_bc):
    mesh = plsc.VectorSubcoreMesh(core_axis_name="c", subcore_axis_name="s")
    f = pl.kernel(
        _scatter_body,
        out_type=jax.ShapeDtypeStruct((2, NK, DP), jnp.float32),
        mesh=mesh,
        scratch_types=[pltpu.VMEM((G, GSZ), jnp.int32),
                       pltpu.VMEM((RPW, DP), jnp.float32),
                       pltpu.VMEM((RPT, DP), jnp.float32),
                       pltpu.VMEM((RPT, CW), jnp.float32),
                       pltpu.VMEM_SHARED((NK, DP), jnp.float32),
                       pltpu.SemaphoreType.DMA,
                       pltpu.SemaphoreType.DMA],
    )
    return f(xt, idx_g, zdw, emaw_p, ecs_bc)


def _ema_gather_body(dwp_hbm, n_hbm, idx_hbm, emaw_hbm, q_hbm, parts_hbm,
                     idx_v, rows_v, dw0_v, dw1_v, emaw_v, n_row, wnew_v,
                     parts_v, wnew_sh, sem, wsem):
    c = lax.axis_index("c")
    s = lax.axis_index("s")
    w = s * 2 + c
    base = s * RPT
    pltpu.sync_copy(idx_hbm.at[w], idx_v)
    pltpu.sync_copy(dwp_hbm.at[0, pl.ds(base, RPT)], dw0_v)
    pltpu.sync_copy(dwp_hbm.at[1, pl.ds(base, RPT)], dw1_v)
    pltpu.sync_copy(emaw_hbm.at[pl.ds(base, RPT)], emaw_v)
    pltpu.sync_copy(n_hbm.at[pl.ds(0, 1)], n_row)
    nv = n_row[0, pl.ds(0, 16)]
    den = nv + NK * EPS
    OM = 1.0 - DECAY

    def _row_step(r, carry):
        vqx, vq2 = carry
        cnt = dw0_v[r, pl.ds(D, 16)] + dw1_v[r, pl.ds(D, 16)]
        ncs = OM * cnt
        cs = (ncs + EPS) / den * nv
        iv = OM / cs
        w2r = jnp.zeros((16,), jnp.float32)
        for k in range(D // 16):
            sl = pl.ds(k * 16, 16)
            acc = dw0_v[r, sl] + dw1_v[r, sl]
            wn = acc * iv
            wnew_v[r, sl] = wn
            vqx = vqx + wn * (acc - RATIO * emaw_v[r, sl])
            w2r = w2r + wn * wn
        vq2 = vq2 + cnt * w2r
        return (vqx, vq2)

    vqx, vq2 = lax.fori_loop(0, RPT, _row_step,
                             (jnp.zeros((16,), jnp.float32),
                              jnp.zeros((16,), jnp.float32)))
    parts_v[0, pl.ds(0, 16)] = vqx
    parts_v[1, pl.ds(0, 16)] = vq2
    pltpu.sync_copy(parts_v, parts_hbm.at[w])
    pltpu.sync_copy(wnew_v, wnew_sh.at[pl.ds(base, RPT)])
    plsc.subcore_barrier()

    cps = [pltpu.async_copy(wnew_sh.at[idx_v.at[g]],
                            rows_v.at[pl.ds(g * GSZ, GSZ)], sem)
           for g in range(G)]
    wcps = []
    for g in range(G):
        cps[g].wait()
        wcps.append(pltpu.async_copy(
            rows_v.at[pl.ds(g * GSZ, GSZ)],
            q_hbm.at[w // 2, pl.ds((w % 2) * RPW + g * GSZ, GSZ)], wsem))
    for cp in wcps:
        cp.wait()


def _stage_c(dwp, nsum, idx_g, emaw_p):
    mesh = plsc.VectorSubcoreMesh(core_axis_name="c", subcore_axis_name="s")
    f = pl.kernel(
        _ema_gather_body,
        out_type=[jax.ShapeDtypeStruct((B, PIX, DP), jnp.float32),
                  jax.ShapeDtypeStruct((NW, 2, 16), jnp.float32)],
        mesh=mesh,
        scratch_types=[pltpu.VMEM((G, GSZ), jnp.int32),
                       pltpu.VMEM((RPW, DP), jnp.float32),
                       pltpu.VMEM((RPT, DP), jnp.float32),
                       pltpu.VMEM((RPT, DP), jnp.float32),
                       pltpu.VMEM((RPT, DP), jnp.float32),
                       pltpu.VMEM((1, DP), jnp.float32),
                       pltpu.VMEM((RPT, DP), jnp.float32),
                       pltpu.VMEM((2, 16), jnp.float32),
                       pltpu.VMEM_SHARED((NK, DP), jnp.float32),
                       pltpu.SemaphoreType.DMA,
                       pltpu.SemaphoreType.DMA],
    )
    return f(dwp, nsum, idx_g, emaw_p)


def _finish_body(n_ref, parts_ref, loss_ref):
    x2tot = n_ref[1, 0]
    qx = 0.5 * jnp.sum(parts_ref[:, 0, :])
    q2 = 0.5 * jnp.sum(parts_ref[:, 1, :])
    loss_ref[0, 0] = (q2 - 2.0 * qx + x2tot) * (CC / (N * D))


def _stage_f(nsum, parts):
    return pl.pallas_call(
        _finish_body,
        in_specs=[pl.BlockSpec(memory_space=pltpu.SMEM),
                  pl.BlockSpec((NW, 2, 16), lambda: (0, 0, 0))],
        out_specs=pl.BlockSpec(memory_space=pltpu.SMEM),
        out_shape=jax.ShapeDtypeStruct((1, 1), jnp.float32),
    )(nsum, parts)


def kernel(inputs, W, ema_cluster_size, ema_w):
    x3 = inputs.reshape(B, D, PIX)
    idx3, xt, nsum = _stage_a(x3, W, ema_cluster_size.reshape(8, DP))
    idx_g = idx3.reshape(NW, G, GSZ)
    zdw = jnp.zeros((NK, DP), jnp.float32)
    emaw_p = jnp.pad(ema_w, ((0, 0), (0, DP - D)))
    ecs_bc = jnp.broadcast_to(ema_cluster_size[:, None], (NK, CW))
    dwp = _stage_b(xt, idx_g, zdw, emaw_p, ecs_bc)
    q3, parts = _stage_c(dwp, nsum, idx_g, emaw_p)
    loss2 = _stage_f(nsum, parts)
    qout = q3[:, :, :D].transpose(0, 2, 1).reshape(B, D, 32, 32)
    return (qout, loss2.reshape(()), idx3.reshape(B, 32, 32))
